# baseline (speedup 1.0000x reference)
import functools

import jax
import jax.numpy as jnp
from jax import lax
from jax.experimental import pallas as pl
from jax.experimental.pallas import tpu as pltpu
from jax.experimental.pallas import tpu_sc as plsc

B = 16384
MAX_F = 32
NUM_F = 768
FT_OUT = 256

NC, NS, L = 2, 16, 16
NW = NC * NS
R = 64
SPLIT = 4
TB = 2048


def _make_sc_body(nb):
    rows_per_w = nb // NW
    ch = rows_per_w // R

    def _sc_body(stm_hbm, nstm_hbm, val_hbm, out_hbm,
                 stm_v, nstm_v, val_v, s_a, s_b, sem_a, sem_b, sem_in):
        wid = lax.axis_index("s") * NC + lax.axis_index("c")
        base_row = wid * rows_per_w

        zero16 = jnp.zeros((L,), jnp.float32)
        ones16 = jnp.ones((L,), jnp.int32)
        rowv0 = jnp.zeros((L,), jnp.int32)

        @plsc.parallel_loop(0, R, unroll=2)
        def _zinit(r):
            for j in range(NUM_F // L):
                s_a[r, pl.ds(j * L, L)] = zero16
                s_b[r, pl.ds(j * L, L)] = zero16

        def _scatter_set(idx_v, s_v):
            @plsc.parallel_loop(0, R, unroll=4, carry=rowv0)
            def _row(r, rowv):
                for j in range(MAX_F // L):
                    iv = idx_v[r, pl.ds(j * L, L)]
                    vv = val_v[r, pl.ds(j * L, L)]
                    plsc.addupdate_scatter(s_v, [rowv, iv], vv)
                return rowv + ones16

        def _zero_set(idx_v, s_v):
            @plsc.parallel_loop(0, R, unroll=4, carry=rowv0)
            def _zrow(r, rowv):
                for j in range(MAX_F // L):
                    iv = idx_v[r, pl.ds(j * L, L)]
                    plsc.store_scatter(s_v, [rowv, iv], zero16)
                return rowv + ones16

        def _stage(brow0):
            ca = pltpu.make_async_copy(stm_hbm.at[pl.ds(brow0, R), :], stm_v, sem_in)
            cb = pltpu.make_async_copy(nstm_hbm.at[pl.ds(brow0, R), :], nstm_v, sem_in)
            cc = pltpu.make_async_copy(val_hbm.at[pl.ds(brow0, R), :], val_v, sem_in)
            ca.start()
            cb.start()
            cc.start()
            ca.wait()
            cb.wait()
            cc.wait()

        def _chunk(ci, c):
            brow0 = pl.multiple_of(base_row + ci * R, R)
            @pl.when(ci > 0)
            def _():
                pltpu.make_async_copy(s_a, out_hbm.at[pl.ds(brow0 - R, R), :], sem_a).wait()
                _zero_set(stm_v, s_a)
                pltpu.make_async_copy(s_b, out_hbm.at[pl.ds(nb + brow0 - R, R), :], sem_b).wait()
                _zero_set(nstm_v, s_b)
            _stage(brow0)
            _scatter_set(stm_v, s_a)
            pltpu.async_copy(s_a, out_hbm.at[pl.ds(brow0, R), :], sem_a)
            _scatter_set(nstm_v, s_b)
            pltpu.async_copy(s_b, out_hbm.at[pl.ds(nb + brow0, R), :], sem_b)
            return c
        lax.fori_loop(0, ch, _chunk, 0)

        last = pl.multiple_of(base_row + (ch - 1) * R, R)
        pltpu.make_async_copy(s_a, out_hbm.at[pl.ds(last, R), :], sem_a).wait()
        pltpu.make_async_copy(s_b, out_hbm.at[pl.ds(nb + last, R), :], sem_b).wait()

    return _sc_body


@functools.cache
def _sc_scatter(nb):
    return pl.kernel(
        _make_sc_body(nb),
        mesh=plsc.VectorSubcoreMesh(core_axis_name="c", subcore_axis_name="s"),
        compiler_params=pltpu.CompilerParams(needs_layout_passes=False),
        out_type=jax.ShapeDtypeStruct((2 * nb, NUM_F), jnp.float32),
        scratch_types=[
            pltpu.VMEM((R, MAX_F), jnp.int32),
            pltpu.VMEM((R, MAX_F), jnp.int32),
            pltpu.VMEM((R, MAX_F), jnp.float32),
            pltpu.VMEM((R, NUM_F), jnp.float32),
            pltpu.VMEM((R, NUM_F), jnp.float32),
            pltpu.SemaphoreType.DMA,
            pltpu.SemaphoreType.DMA,
            pltpu.SemaphoreType.DMA,
        ],
    )


def _tc_body(s1_ref, s2_ref, w_ref, b_ref, ow_ref, ob_ref, o_ref):
    w = w_ref[...].astype(jnp.bfloat16)
    h1 = jnp.dot(s1_ref[...].astype(jnp.bfloat16), w,
                 preferred_element_type=jnp.float32) + b_ref[...]
    h2 = jnp.dot(s2_ref[...].astype(jnp.bfloat16), w,
                 preferred_element_type=jnp.float32) + b_ref[...]
    h1 = jnp.clip(h1, 0.0, 1.0)
    h2 = jnp.clip(h2, 0.0, 1.0)
    ow = ow_ref[...]
    acc = (jnp.sum(h1 * ow[:, :FT_OUT], axis=1, keepdims=True)
           + jnp.sum(h2 * ow[:, FT_OUT:], axis=1, keepdims=True))
    o_ref[...] = jax.nn.sigmoid(acc + ob_ref[...])


def _tc_head(s, nb, ft_weight, ft_bias, out_weight, out_bias):
    tb = min(TB, nb)
    grid = nb // tb
    return pl.pallas_call(
        _tc_body,
        grid=(grid,),
        in_specs=[
            pl.BlockSpec((tb, NUM_F), lambda i: (i, 0)),
            pl.BlockSpec((tb, NUM_F), lambda i: (nb // tb + i, 0)),
            pl.BlockSpec((NUM_F, FT_OUT), lambda i: (0, 0)),
            pl.BlockSpec((1, FT_OUT), lambda i: (0, 0)),
            pl.BlockSpec((1, 2 * FT_OUT), lambda i: (0, 0)),
            pl.BlockSpec((1, 1), lambda i: (0, 0)),
        ],
        out_specs=pl.BlockSpec((tb, 1), lambda i: (i, 0)),
        out_shape=jax.ShapeDtypeStruct((nb, 1), jnp.float32),
    )(s, s, ft_weight, ft_bias.reshape(1, FT_OUT),
      out_weight.reshape(1, 2 * FT_OUT), out_bias.reshape(1, 1))


def kernel(values, stm_indices, nstm_indices, ft_weight, ft_bias, out_weight, out_bias):
    values = values.reshape(B, MAX_F).astype(jnp.float32)
    stm = stm_indices.reshape(B, MAX_F).astype(jnp.int32)
    nstm = nstm_indices.reshape(B, MAX_F).astype(jnp.int32)

    nb = B // SPLIT
    outs = []
    for k in range(SPLIT):
        lo = k * nb
        s = _sc_scatter(nb)(stm[lo:lo + nb], nstm[lo:lo + nb], values[lo:lo + nb])
        outs.append(_tc_head(s, nb, ft_weight, ft_bias, out_weight, out_bias))
    return jnp.concatenate(outs, axis=0) if SPLIT > 1 else outs[0]


# speedup vs baseline: 1.0900x; 1.0900x over previous
import functools

import jax
import jax.numpy as jnp
from jax import lax
from jax.experimental import pallas as pl
from jax.experimental.pallas import tpu as pltpu
from jax.experimental.pallas import tpu_sc as plsc

B = 16384
MAX_F = 32
NUM_F = 768
FT_OUT = 256

NC, NS, L = 2, 16, 16
NW = NC * NS
R = 64
SPLIT = 2
TB = 2048


def _make_sc_body(nb):
    rows_per_w = nb // NW
    ch = rows_per_w // R

    def _sc_body(stm_hbm, nstm_hbm, val_hbm, out_hbm,
                 stm_v, nstm_v, val_v, s_a, s_b, sem_a, sem_b, sem_in):
        wid = lax.axis_index("s") * NC + lax.axis_index("c")
        base_row = wid * rows_per_w

        zero16 = jnp.zeros((L,), jnp.float32)
        ones16 = jnp.ones((L,), jnp.int32)
        rowv0 = jnp.zeros((L,), jnp.int32)

        @plsc.parallel_loop(0, R, unroll=2)
        def _zinit(r):
            for j in range(NUM_F // L):
                s_a[r, pl.ds(j * L, L)] = zero16
                s_b[r, pl.ds(j * L, L)] = zero16

        def _scatter_set(idx_v, s_v):
            @plsc.parallel_loop(0, R, unroll=4, carry=rowv0)
            def _row(r, rowv):
                for j in range(MAX_F // L):
                    iv = idx_v[r, pl.ds(j * L, L)]
                    vv = val_v[r, pl.ds(j * L, L)]
                    plsc.addupdate_scatter(s_v, [rowv, iv], vv)
                return rowv + ones16

        def _zero_set(idx_v, s_v):
            @plsc.parallel_loop(0, R, unroll=4, carry=rowv0)
            def _zrow(r, rowv):
                for j in range(MAX_F // L):
                    iv = idx_v[r, pl.ds(j * L, L)]
                    plsc.store_scatter(s_v, [rowv, iv], zero16)
                return rowv + ones16

        def _stage(brow0):
            ca = pltpu.make_async_copy(stm_hbm.at[pl.ds(brow0, R), :], stm_v, sem_in)
            cb = pltpu.make_async_copy(nstm_hbm.at[pl.ds(brow0, R), :], nstm_v, sem_in)
            cc = pltpu.make_async_copy(val_hbm.at[pl.ds(brow0, R), :], val_v, sem_in)
            ca.start()
            cb.start()
            cc.start()
            ca.wait()
            cb.wait()
            cc.wait()

        def _chunk(ci, c):
            brow0 = pl.multiple_of(base_row + ci * R, R)
            @pl.when(ci > 0)
            def _():
                pltpu.make_async_copy(s_a, out_hbm.at[pl.ds(brow0 - R, R), :], sem_a).wait()
                _zero_set(stm_v, s_a)
                pltpu.make_async_copy(s_b, out_hbm.at[pl.ds(nb + brow0 - R, R), :], sem_b).wait()
                _zero_set(nstm_v, s_b)
            _stage(brow0)
            _scatter_set(stm_v, s_a)
            pltpu.async_copy(s_a, out_hbm.at[pl.ds(brow0, R), :], sem_a)
            _scatter_set(nstm_v, s_b)
            pltpu.async_copy(s_b, out_hbm.at[pl.ds(nb + brow0, R), :], sem_b)
            return c
        lax.fori_loop(0, ch, _chunk, 0)

        last = pl.multiple_of(base_row + (ch - 1) * R, R)
        pltpu.make_async_copy(s_a, out_hbm.at[pl.ds(last, R), :], sem_a).wait()
        pltpu.make_async_copy(s_b, out_hbm.at[pl.ds(nb + last, R), :], sem_b).wait()

    return _sc_body


@functools.cache
def _sc_scatter(nb):
    return pl.kernel(
        _make_sc_body(nb),
        mesh=plsc.VectorSubcoreMesh(core_axis_name="c", subcore_axis_name="s"),
        compiler_params=pltpu.CompilerParams(needs_layout_passes=False),
        out_type=jax.ShapeDtypeStruct((2 * nb, NUM_F), jnp.float32),
        scratch_types=[
            pltpu.VMEM((R, MAX_F), jnp.int32),
            pltpu.VMEM((R, MAX_F), jnp.int32),
            pltpu.VMEM((R, MAX_F), jnp.float32),
            pltpu.VMEM((R, NUM_F), jnp.float32),
            pltpu.VMEM((R, NUM_F), jnp.float32),
            pltpu.SemaphoreType.DMA,
            pltpu.SemaphoreType.DMA,
            pltpu.SemaphoreType.DMA,
        ],
    )


def _tc_body(s1_ref, s2_ref, w_ref, b_ref, ow_ref, ob_ref, o_ref):
    w = w_ref[...].astype(jnp.bfloat16)
    h1 = jnp.dot(s1_ref[...].astype(jnp.bfloat16), w,
                 preferred_element_type=jnp.float32) + b_ref[...]
    h2 = jnp.dot(s2_ref[...].astype(jnp.bfloat16), w,
                 preferred_element_type=jnp.float32) + b_ref[...]
    h1 = jnp.clip(h1, 0.0, 1.0)
    h2 = jnp.clip(h2, 0.0, 1.0)
    ow = ow_ref[...]
    acc = (jnp.sum(h1 * ow[:, :FT_OUT], axis=1, keepdims=True)
           + jnp.sum(h2 * ow[:, FT_OUT:], axis=1, keepdims=True))
    o_ref[...] = jax.nn.sigmoid(acc + ob_ref[...])


def _tc_head(s, nb, ft_weight, ft_bias, out_weight, out_bias):
    tb = min(TB, nb)
    grid = nb // tb
    return pl.pallas_call(
        _tc_body,
        grid=(grid,),
        in_specs=[
            pl.BlockSpec((tb, NUM_F), lambda i: (i, 0)),
            pl.BlockSpec((tb, NUM_F), lambda i: (nb // tb + i, 0)),
            pl.BlockSpec((NUM_F, FT_OUT), lambda i: (0, 0)),
            pl.BlockSpec((1, FT_OUT), lambda i: (0, 0)),
            pl.BlockSpec((1, 2 * FT_OUT), lambda i: (0, 0)),
            pl.BlockSpec((1, 1), lambda i: (0, 0)),
        ],
        out_specs=pl.BlockSpec((tb, 1), lambda i: (i, 0)),
        out_shape=jax.ShapeDtypeStruct((nb, 1), jnp.float32),
    )(s, s, ft_weight, ft_bias.reshape(1, FT_OUT),
      out_weight.reshape(1, 2 * FT_OUT), out_bias.reshape(1, 1))


def kernel(values, stm_indices, nstm_indices, ft_weight, ft_bias, out_weight, out_bias):
    values = values.reshape(B, MAX_F).astype(jnp.float32)
    stm = stm_indices.reshape(B, MAX_F).astype(jnp.int32)
    nstm = nstm_indices.reshape(B, MAX_F).astype(jnp.int32)

    nb = B // SPLIT
    outs = []
    for k in range(SPLIT):
        lo = k * nb
        s = _sc_scatter(nb)(stm[lo:lo + nb], nstm[lo:lo + nb], values[lo:lo + nb])
        outs.append(_tc_head(s, nb, ft_weight, ft_bias, out_weight, out_bias))
    return jnp.concatenate(outs, axis=0) if SPLIT > 1 else outs[0]


# speedup vs baseline: 1.0910x; 1.0009x over previous
import functools

import jax
import jax.numpy as jnp
from jax import lax
from jax.experimental import pallas as pl
from jax.experimental.pallas import tpu as pltpu
from jax.experimental.pallas import tpu_sc as plsc

B = 16384
MAX_F = 32
NUM_F = 768
FT_OUT = 256

NC, NS, L = 2, 16, 16
NW = NC * NS
R = 64
SEGS = (10240, 6144)
TB = 2048


def _make_sc_body(nb):
    rows_per_w = nb // NW
    ch = rows_per_w // R

    def _sc_body(stm_hbm, nstm_hbm, val_hbm, out_hbm,
                 stm_v, nstm_v, val_v, s_a, s_b, sem_a, sem_b, sem_in):
        wid = lax.axis_index("s") * NC + lax.axis_index("c")
        base_row = wid * rows_per_w

        zero16 = jnp.zeros((L,), jnp.float32)
        ones16 = jnp.ones((L,), jnp.int32)
        rowv0 = jnp.zeros((L,), jnp.int32)

        @plsc.parallel_loop(0, R, unroll=2)
        def _zinit(r):
            for j in range(NUM_F // L):
                s_a[r, pl.ds(j * L, L)] = zero16
                s_b[r, pl.ds(j * L, L)] = zero16

        def _scatter_set(idx_v, s_v):
            @plsc.parallel_loop(0, R, unroll=4, carry=rowv0)
            def _row(r, rowv):
                for j in range(MAX_F // L):
                    iv = idx_v[r, pl.ds(j * L, L)]
                    vv = val_v[r, pl.ds(j * L, L)]
                    plsc.addupdate_scatter(s_v, [rowv, iv], vv)
                return rowv + ones16

        def _zero_set(idx_v, s_v):
            @plsc.parallel_loop(0, R, unroll=4, carry=rowv0)
            def _zrow(r, rowv):
                for j in range(MAX_F // L):
                    iv = idx_v[r, pl.ds(j * L, L)]
                    plsc.store_scatter(s_v, [rowv, iv], zero16)
                return rowv + ones16

        def _stage(brow0):
            ca = pltpu.make_async_copy(stm_hbm.at[pl.ds(brow0, R), :], stm_v, sem_in)
            cb = pltpu.make_async_copy(nstm_hbm.at[pl.ds(brow0, R), :], nstm_v, sem_in)
            cc = pltpu.make_async_copy(val_hbm.at[pl.ds(brow0, R), :], val_v, sem_in)
            ca.start()
            cb.start()
            cc.start()
            ca.wait()
            cb.wait()
            cc.wait()

        def _chunk(ci, c):
            brow0 = pl.multiple_of(base_row + ci * R, R)
            @pl.when(ci > 0)
            def _():
                pltpu.make_async_copy(s_a, out_hbm.at[pl.ds(brow0 - R, R), :], sem_a).wait()
                _zero_set(stm_v, s_a)
                pltpu.make_async_copy(s_b, out_hbm.at[pl.ds(nb + brow0 - R, R), :], sem_b).wait()
                _zero_set(nstm_v, s_b)
            _stage(brow0)
            _scatter_set(stm_v, s_a)
            pltpu.async_copy(s_a, out_hbm.at[pl.ds(brow0, R), :], sem_a)
            _scatter_set(nstm_v, s_b)
            pltpu.async_copy(s_b, out_hbm.at[pl.ds(nb + brow0, R), :], sem_b)
            return c
        lax.fori_loop(0, ch, _chunk, 0)

        last = pl.multiple_of(base_row + (ch - 1) * R, R)
        pltpu.make_async_copy(s_a, out_hbm.at[pl.ds(last, R), :], sem_a).wait()
        pltpu.make_async_copy(s_b, out_hbm.at[pl.ds(nb + last, R), :], sem_b).wait()

    return _sc_body


@functools.cache
def _sc_scatter(nb):
    return pl.kernel(
        _make_sc_body(nb),
        mesh=plsc.VectorSubcoreMesh(core_axis_name="c", subcore_axis_name="s"),
        compiler_params=pltpu.CompilerParams(needs_layout_passes=False),
        out_type=jax.ShapeDtypeStruct((2 * nb, NUM_F), jnp.float32),
        scratch_types=[
            pltpu.VMEM((R, MAX_F), jnp.int32),
            pltpu.VMEM((R, MAX_F), jnp.int32),
            pltpu.VMEM((R, MAX_F), jnp.float32),
            pltpu.VMEM((R, NUM_F), jnp.float32),
            pltpu.VMEM((R, NUM_F), jnp.float32),
            pltpu.SemaphoreType.DMA,
            pltpu.SemaphoreType.DMA,
            pltpu.SemaphoreType.DMA,
        ],
    )


def _tc_body(s1_ref, s2_ref, w_ref, b_ref, ow_ref, ob_ref, o_ref):
    w = w_ref[...].astype(jnp.bfloat16)
    h1 = jnp.dot(s1_ref[...].astype(jnp.bfloat16), w,
                 preferred_element_type=jnp.float32) + b_ref[...]
    h2 = jnp.dot(s2_ref[...].astype(jnp.bfloat16), w,
                 preferred_element_type=jnp.float32) + b_ref[...]
    h1 = jnp.clip(h1, 0.0, 1.0)
    h2 = jnp.clip(h2, 0.0, 1.0)
    ow = ow_ref[...]
    acc = (jnp.sum(h1 * ow[:, :FT_OUT], axis=1, keepdims=True)
           + jnp.sum(h2 * ow[:, FT_OUT:], axis=1, keepdims=True))
    o_ref[...] = jax.nn.sigmoid(acc + ob_ref[...])


def _tc_head(s, nb, ft_weight, ft_bias, out_weight, out_bias):
    tb = min(TB, nb)
    grid = nb // tb
    return pl.pallas_call(
        _tc_body,
        grid=(grid,),
        in_specs=[
            pl.BlockSpec((tb, NUM_F), lambda i: (i, 0)),
            pl.BlockSpec((tb, NUM_F), lambda i: (nb // tb + i, 0)),
            pl.BlockSpec((NUM_F, FT_OUT), lambda i: (0, 0)),
            pl.BlockSpec((1, FT_OUT), lambda i: (0, 0)),
            pl.BlockSpec((1, 2 * FT_OUT), lambda i: (0, 0)),
            pl.BlockSpec((1, 1), lambda i: (0, 0)),
        ],
        out_specs=pl.BlockSpec((tb, 1), lambda i: (i, 0)),
        out_shape=jax.ShapeDtypeStruct((nb, 1), jnp.float32),
    )(s, s, ft_weight, ft_bias.reshape(1, FT_OUT),
      out_weight.reshape(1, 2 * FT_OUT), out_bias.reshape(1, 1))


def kernel(values, stm_indices, nstm_indices, ft_weight, ft_bias, out_weight, out_bias):
    values = values.reshape(B, MAX_F).astype(jnp.float32)
    stm = stm_indices.reshape(B, MAX_F).astype(jnp.int32)
    nstm = nstm_indices.reshape(B, MAX_F).astype(jnp.int32)

    outs = []
    lo = 0
    for nb in SEGS:
        s = _sc_scatter(nb)(stm[lo:lo + nb], nstm[lo:lo + nb], values[lo:lo + nb])
        outs.append(_tc_head(s, nb, ft_weight, ft_bias, out_weight, out_bias))
        lo += nb
    return jnp.concatenate(outs, axis=0) if len(SEGS) > 1 else outs[0]
